# baseline (speedup 1.0000x reference)
import jax
import jax.numpy as jnp
from jax.experimental import pallas as pl


def _copy_body(x_ref, o_ref):
    o_ref[...] = x_ref[...]


def kernel(t_obs, s_obs, states, log_weights, onsets, sigma, noise_eps, u_samples, W1, b1, W2, b2, W3, b3):
    d = states.shape[1]
    Bn = t_obs.shape[0]
    mean_loglik = jnp.zeros((Bn,), dtype=t_obs.dtype)
    std_loglik = jnp.zeros((Bn,), dtype=t_obs.dtype)
    ess = jnp.ones((Bn,), dtype=t_obs.dtype)
    x = jnp.stack([t_obs / 100.0, s_obs, jnp.tanh(mean_loglik / 50.0), jnp.tanh(std_loglik / 10.0), ess], axis=-1)
    h = jax.nn.relu(x @ W1 + b1)
    h = jax.nn.relu(h @ W2 + b2)
    out = jax.nn.softplus(h @ W3 + b3)
    out_mean = out.mean(axis=0)
    noise_scale = out_mean[:d]
    correction = out_mean[d:]
    correct_prior = correction[:d]
    correct_lik = correction[-2]
    forget_lik = correction[-1]
    new_states = states + noise_eps * sigma[None, :] * noise_scale[None, :]
    rate = jax.nn.softplus(new_states[:, 0:1])
    dt = jnp.maximum(t_obs[None, :] - onsets[:, None], 0.0)
    s_pred = jnp.exp(-rate * dt)
    loglik = -0.5 * jnp.sum((s_obs[None, :] - s_pred) ** 2, axis=1)
    prior_mean = jnp.mean(states, axis=0)
    prior_term = -0.5 * jnp.sum(correct_prior[None, :] * (new_states - prior_mean[None, :]) ** 2, axis=1)
    new_logw = forget_lik * log_weights + correct_lik * loglik + prior_term
    weights = jax.nn.softmax(new_logw)
    cdf = jnp.cumsum(weights)
    idx = jnp.clip(jnp.searchsorted(cdf, u_samples), 0, weights.shape[0] - 1)
    resampled = jnp.take(new_states, idx, axis=0)
    out = pl.pallas_call(
        _copy_body,
        out_shape=jax.ShapeDtypeStruct(resampled.shape, resampled.dtype),
        grid=(resampled.shape[0] // 8192,),
        in_specs=[pl.BlockSpec((8192, d), lambda i: (i, 0))],
        out_specs=pl.BlockSpec((8192, d), lambda i: (i, 0)),
    )(resampled)
    return out


# speedup vs baseline: 5.2954x; 5.2954x over previous
import dataclasses
import functools

import jax
import jax.numpy as jnp
from jax import lax
from jax.experimental import pallas as pl
from jax.experimental.pallas import tpu as pltpu
from jax.experimental.pallas import tpu_sc as plsc

N = 524288
D = 8
NUM_WORKERS = 32
Q_PER_WORKER = N // NUM_WORKERS
BQ = 128
NUM_BLOCKS = Q_PER_WORKER // BQ
CHUNK = 8
NUM_CHUNKS = N // CHUNK
LANES = 16


def _sc_compiler_params():
    cp = pltpu.CompilerParams()
    fields = pltpu.CompilerParams.__dataclass_fields__
    if "needs_layout_passes" in fields:
        cp = dataclasses.replace(cp, needs_layout_passes=False)
    if "use_tc_tiling_on_sc" in fields:
        cp = dataclasses.replace(cp, use_tc_tiling_on_sc=False)
    return cp


def _resample_sc(cdf8, chunk_cdf, u_samples, new_states):
    mesh = plsc.VectorSubcoreMesh(core_axis_name="c", subcore_axis_name="s")

    @functools.partial(
        pl.kernel,
        out_type=jax.ShapeDtypeStruct((N, D), jnp.float32),
        mesh=mesh,
        compiler_params=_sc_compiler_params(),
        scratch_types=[
            pltpu.VMEM((NUM_CHUNKS,), jnp.float32),
            pltpu.VMEM((BQ,), jnp.float32),
            pltpu.VMEM((BQ,), jnp.int32),
            pltpu.VMEM((BQ, CHUNK), jnp.float32),
            pltpu.VMEM((BQ,), jnp.int32),
            pltpu.VMEM((BQ, D), jnp.float32),
        ],
    )
    def k(cdf8_hbm, t_hbm, u_hbm, ns_hbm, out_hbm, t_v, u_v, c_v, rows_v, i_v, o_v):
        wid = lax.axis_index("s") * 2 + lax.axis_index("c")
        base_q = wid * Q_PER_WORKER
        pltpu.sync_copy(t_hbm, t_v)

        @pl.loop(0, NUM_BLOCKS)
        def _(blk):
            qb = base_q + blk * BQ
            pltpu.sync_copy(u_hbm.at[pl.ds(qb, BQ)], u_v)

            @pl.loop(0, BQ, step=LANES)
            def _(voff):
                u16 = u_v[pl.ds(voff, LANES)]
                pos = jnp.zeros((LANES,), jnp.int32)
                for bit in [1 << b for b in range(15, -1, -1)]:
                    probe = pos + (bit - 1)
                    val = plsc.load_gather(t_v, [probe])
                    pos = pos + jnp.where(val < u16, jnp.int32(bit), jnp.int32(0))
                c_v[pl.ds(voff, LANES)] = pos

            pltpu.sync_copy(cdf8_hbm.at[c_v], rows_v)

            @pl.loop(0, BQ, step=LANES)
            def _(voff):
                u16 = u_v[pl.ds(voff, LANES)]
                c16 = c_v[pl.ds(voff, LANES)]
                row = voff + lax.iota(jnp.int32, LANES)
                cnt = jnp.zeros((LANES,), jnp.int32)
                for kk in range(CHUNK):
                    col = jnp.full((LANES,), kk, jnp.int32)
                    vals = plsc.load_gather(rows_v, [row, col])
                    cnt = cnt + jnp.where(vals < u16, jnp.int32(1), jnp.int32(0))
                i_v[pl.ds(voff, LANES)] = jnp.minimum(c16 * CHUNK + cnt, jnp.int32(N - 1))

            pltpu.sync_copy(ns_hbm.at[i_v], o_v)
            pltpu.sync_copy(o_v, out_hbm.at[pl.ds(qb, BQ)])

    return k(cdf8, chunk_cdf, u_samples, new_states)


def kernel(t_obs, s_obs, states, log_weights, onsets, sigma, noise_eps, u_samples, W1, b1, W2, b2, W3, b3):
    d = states.shape[1]
    Bn = t_obs.shape[0]
    mean_loglik = jnp.zeros((Bn,), dtype=t_obs.dtype)
    std_loglik = jnp.zeros((Bn,), dtype=t_obs.dtype)
    ess = jnp.ones((Bn,), dtype=t_obs.dtype)
    x = jnp.stack([t_obs / 100.0, s_obs, jnp.tanh(mean_loglik / 50.0), jnp.tanh(std_loglik / 10.0), ess], axis=-1)
    h = jax.nn.relu(x @ W1 + b1)
    h = jax.nn.relu(h @ W2 + b2)
    out = jax.nn.softplus(h @ W3 + b3)
    out_mean = out.mean(axis=0)
    noise_scale = out_mean[:d]
    correction = out_mean[d:]
    correct_prior = correction[:d]
    correct_lik = correction[-2]
    forget_lik = correction[-1]
    new_states = states + noise_eps * sigma[None, :] * noise_scale[None, :]
    rate = jax.nn.softplus(new_states[:, 0:1])
    dt = jnp.maximum(t_obs[None, :] - onsets[:, None], 0.0)
    s_pred = jnp.exp(-rate * dt)
    loglik = -0.5 * jnp.sum((s_obs[None, :] - s_pred) ** 2, axis=1)
    prior_mean = jnp.mean(states, axis=0)
    prior_term = -0.5 * jnp.sum(correct_prior[None, :] * (new_states - prior_mean[None, :]) ** 2, axis=1)
    new_logw = forget_lik * log_weights + correct_lik * loglik + prior_term
    weights = jax.nn.softmax(new_logw)
    cdf = jnp.cumsum(weights)
    cdf8 = cdf.reshape(NUM_CHUNKS, CHUNK)
    chunk_cdf = cdf8[:, CHUNK - 1]
    return _resample_sc(cdf8, chunk_cdf, u_samples, new_states)


# speedup vs baseline: 6.4485x; 1.2178x over previous
import dataclasses
import functools

import jax
import jax.numpy as jnp
from jax import lax
from jax.experimental import pallas as pl
from jax.experimental.pallas import tpu as pltpu
from jax.experimental.pallas import tpu_sc as plsc

N = 524288
D = 8
NUM_WORKERS = 32
Q_PER_WORKER = N // NUM_WORKERS
SB = 512
NSB = Q_PER_WORKER // SB
NJ = SB // 128
CHUNK = 8
NUM_CHUNKS = N // CHUNK
LANES = 16


def _sc_compiler_params():
    cp = pltpu.CompilerParams()
    fields = pltpu.CompilerParams.__dataclass_fields__
    if "needs_layout_passes" in fields:
        cp = dataclasses.replace(cp, needs_layout_passes=False)
    if "use_tc_tiling_on_sc" in fields:
        cp = dataclasses.replace(cp, use_tc_tiling_on_sc=False)
    return cp


def _resample_sc(cdf8, chunk_cdf, u_samples, new_states):
    mesh = plsc.VectorSubcoreMesh(core_axis_name="c", subcore_axis_name="s")

    @functools.partial(
        pl.kernel,
        out_type=jax.ShapeDtypeStruct((N, D), jnp.float32),
        mesh=mesh,
        compiler_params=_sc_compiler_params(),
        scratch_types=[
            pltpu.VMEM((NUM_CHUNKS,), jnp.float32),
            pltpu.VMEM((2, SB), jnp.float32),
            pltpu.VMEM((NJ, 128), jnp.int32),
            pltpu.VMEM((SB, CHUNK), jnp.float32),
            pltpu.VMEM((NJ, 128), jnp.int32),
            pltpu.VMEM((2, SB, D), jnp.float32),
            pltpu.SemaphoreType.DMA,
            pltpu.SemaphoreType.DMA,
            pltpu.SemaphoreType.DMA,
            pltpu.SemaphoreType.DMA,
            pltpu.SemaphoreType.DMA,
            pltpu.SemaphoreType.DMA,
        ],
    )
    def k(cdf8_hbm, t_hbm, u_hbm, ns_hbm, out_hbm,
          t_v, u_v, c_v, rows_v, i_v, o_v, su0, su1, sg, sn, so0, so1):
        wid = lax.axis_index("s") * 2 + lax.axis_index("c")
        base_q = wid * Q_PER_WORKER
        su = (su0, su1)
        so = (so0, so1)
        pltpu.sync_copy(t_hbm, t_v)
        pltpu.async_copy(u_hbm.at[pl.ds(base_q, SB)], u_v.at[0], su0)

        @pl.loop(0, NSB // 2)
        def _(g):
            for par in (0, 1):
                s = g * 2 + par
                qb = base_q + s * SB

                @pl.when(g >= 1)
                def _():
                    pltpu.make_async_copy(
                        o_v.at[par], out_hbm.at[pl.ds(base_q, SB)], so[par]
                    ).wait()

                pltpu.make_async_copy(
                    u_hbm.at[pl.ds(base_q, SB)], u_v.at[par], su[par]
                ).wait()

                @pl.when(s + 1 < NSB)
                def _():
                    pltpu.async_copy(
                        u_hbm.at[pl.ds(qb + SB, SB)], u_v.at[1 - par], su[1 - par]
                    )

                g_handles = []
                for j in range(NJ):
                    @pl.loop(0, 128, step=LANES)
                    def _(voff, j=j):
                        u16 = u_v[par, pl.ds(j * 128 + voff, LANES)]
                        pos = jnp.zeros((LANES,), jnp.int32)
                        for bit in [1 << b for b in range(15, -1, -1)]:
                            probe = pos + (bit - 1)
                            val = plsc.load_gather(t_v, [probe])
                            pos = pos + jnp.where(val < u16, jnp.int32(bit), jnp.int32(0))
                        c_v[j, pl.ds(voff, LANES)] = pos

                    g_handles.append(
                        pltpu.async_copy(
                            cdf8_hbm.at[c_v.at[j]],
                            rows_v.at[pl.ds(j * 128, 128)],
                            sg,
                        )
                    )

                n_handles = []
                for j in range(NJ):
                    g_handles[j].wait()

                    @pl.loop(0, 128, step=LANES)
                    def _(voff, j=j):
                        u16 = u_v[par, pl.ds(j * 128 + voff, LANES)]
                        c16 = c_v[j, pl.ds(voff, LANES)]
                        row = j * 128 + voff + lax.iota(jnp.int32, LANES)
                        cnt = jnp.zeros((LANES,), jnp.int32)
                        for kk in range(CHUNK - 1):
                            col = jnp.full((LANES,), kk, jnp.int32)
                            vals = plsc.load_gather(rows_v, [row, col])
                            cnt = cnt + jnp.where(vals < u16, jnp.int32(1), jnp.int32(0))
                        i_v[j, pl.ds(voff, LANES)] = jnp.minimum(
                            c16 * CHUNK + cnt, jnp.int32(N - 1)
                        )

                    n_handles.append(
                        pltpu.async_copy(
                            ns_hbm.at[i_v.at[j]],
                            o_v.at[par, pl.ds(j * 128, 128)],
                            sn,
                        )
                    )

                for j in range(NJ):
                    n_handles[j].wait()
                pltpu.async_copy(o_v.at[par], out_hbm.at[pl.ds(qb, SB)], so[par])

        pltpu.make_async_copy(o_v.at[0], out_hbm.at[pl.ds(base_q, SB)], so0).wait()
        pltpu.make_async_copy(o_v.at[1], out_hbm.at[pl.ds(base_q, SB)], so1).wait()

    return k(cdf8, chunk_cdf, u_samples, new_states)


def kernel(t_obs, s_obs, states, log_weights, onsets, sigma, noise_eps, u_samples, W1, b1, W2, b2, W3, b3):
    d = states.shape[1]
    Bn = t_obs.shape[0]
    mean_loglik = jnp.zeros((Bn,), dtype=t_obs.dtype)
    std_loglik = jnp.zeros((Bn,), dtype=t_obs.dtype)
    ess = jnp.ones((Bn,), dtype=t_obs.dtype)
    x = jnp.stack([t_obs / 100.0, s_obs, jnp.tanh(mean_loglik / 50.0), jnp.tanh(std_loglik / 10.0), ess], axis=-1)
    h = jax.nn.relu(x @ W1 + b1)
    h = jax.nn.relu(h @ W2 + b2)
    out = jax.nn.softplus(h @ W3 + b3)
    out_mean = out.mean(axis=0)
    noise_scale = out_mean[:d]
    correction = out_mean[d:]
    correct_prior = correction[:d]
    correct_lik = correction[-2]
    forget_lik = correction[-1]
    new_states = states + noise_eps * sigma[None, :] * noise_scale[None, :]
    rate = jax.nn.softplus(new_states[:, 0:1])
    dt = jnp.maximum(t_obs[None, :] - onsets[:, None], 0.0)
    s_pred = jnp.exp(-rate * dt)
    loglik = -0.5 * jnp.sum((s_obs[None, :] - s_pred) ** 2, axis=1)
    prior_mean = jnp.mean(states, axis=0)
    prior_term = -0.5 * jnp.sum(correct_prior[None, :] * (new_states - prior_mean[None, :]) ** 2, axis=1)
    new_logw = forget_lik * log_weights + correct_lik * loglik + prior_term
    weights = jax.nn.softmax(new_logw)
    cdf = jnp.cumsum(weights)
    cdf8 = cdf.reshape(NUM_CHUNKS, CHUNK)
    chunk_cdf = cdf8[:, CHUNK - 1]
    return _resample_sc(cdf8, chunk_cdf, u_samples, new_states)
